# Initial kernel scaffold; baseline (speedup 1.0000x reference)
import functools

import jax
import jax.numpy as jnp
from jax import lax
from jax.experimental import pallas as pl
from jax.experimental.pallas import tpu as pltpu
from jax.experimental.pallas import tpu_sc as plsc

N = 10000
E = 320000
D = 128
G = 64

NC = 2
NS = 16
NW = NC * NS
EPW = E // NW
K = 80
NCHUNK = EPW // K
RPS = N // NS
ZR = 125

BN = 1000
NB = N // BN


def _sc_aggregate(x, row, col, w):
    mesh = plsc.VectorSubcoreMesh(core_axis_name="c", subcore_axis_name="s")

    @functools.partial(
        pl.kernel,
        out_type=jax.ShapeDtypeStruct((NC, N, D), jnp.float32),
        mesh=mesh,
        scratch_types=[
            pltpu.VMEM((K,), jnp.int32),
            pltpu.VMEM((K,), jnp.int32),
            pltpu.VMEM((K,), jnp.float32),
            pltpu.VMEM((K, D), jnp.float32),
            pltpu.VMEM((ZR, D), jnp.float32),
            pltpu.VMEM_SHARED((N, D), jnp.float32),
            pltpu.SemaphoreType.DMA,
        ],
    )
    def agg_kernel(x_hbm, row_hbm, col_hbm, w_hbm, out_hbm,
                   row_v, col_v, w_v, rows_v, zero_v, acc_sh, sem):
        cid = lax.axis_index("c")
        sid = lax.axis_index("s")
        wid = sid * NC + cid

        @pl.loop(0, ZR)
        def _(r):
            for dg in range(D // 16):
                zero_v[r, pl.ds(dg * 16, 16)] = jnp.zeros((16,), jnp.float32)

        for j in range(RPS // ZR):
            pltpu.sync_copy(zero_v, acc_sh.at[pl.ds(sid * RPS + j * ZR, ZR)])
        plsc.subcore_barrier()

        base = wid * EPW

        @pl.loop(0, NCHUNK)
        def _(g):
            off = base + g * K
            pltpu.sync_copy(row_hbm.at[pl.ds(off, K)], row_v)
            pltpu.sync_copy(col_hbm.at[pl.ds(off, K)], col_v)
            pltpu.sync_copy(w_hbm.at[pl.ds(off, K)], w_v)
            pltpu.async_copy(x_hbm.at[row_v], rows_v, sem).wait()

            @pl.loop(0, K)
            def _(e):
                wvec = plsc.load_gather(w_v, [jnp.full((16,), e, jnp.int32)])
                for dg in range(D // 16):
                    sl = (e, pl.ds(dg * 16, 16))
                    rows_v[sl] = rows_v[sl] * wvec

            pltpu.sync_copy(rows_v, acc_sh.at[col_v], add=True)

        plsc.subcore_barrier()
        pltpu.sync_copy(acc_sh.at[pl.ds(sid * RPS, RPS)],
                        out_hbm.at[cid].at[pl.ds(sid * RPS, RPS)])

    return agg_kernel(x, row, col, w)


def _tc_transform_body(p_ref, wt_ref, a_ref, b_ref, h_ref, hg_ref):
    i = pl.program_id(0)
    x = p_ref[0] + p_ref[1]
    y = jnp.dot(x, wt_ref[...], preferred_element_type=jnp.float32)
    a = a_ref[0, 0]
    h = jnp.where(y >= 0, y, a * y)
    h_ref[...] = h
    labels = b_ref[0]
    onehot = (lax.broadcasted_iota(jnp.int32, (G, BN), 0) == labels
              ).astype(jnp.float32)
    contrib = jnp.dot(onehot, h, preferred_element_type=jnp.float32)

    @pl.when(i == 0)
    def _():
        hg_ref[...] = jnp.zeros_like(hg_ref)

    hg_ref[...] += contrib


def _tc_transform(parts, wt, a, batch3):
    return pl.pallas_call(
        _tc_transform_body,
        grid=(NB,),
        in_specs=[
            pl.BlockSpec((NC, BN, D), lambda i: (0, i, 0)),
            pl.BlockSpec((D, D), lambda i: (0, 0)),
            pl.BlockSpec((1, 1), lambda i: (0, 0)),
            pl.BlockSpec((1, 1, BN), lambda i: (i, 0, 0)),
        ],
        out_specs=[
            pl.BlockSpec((BN, D), lambda i: (i, 0)),
            pl.BlockSpec((G, D), lambda i: (0, 0)),
        ],
        out_shape=[
            jax.ShapeDtypeStruct((N, D), jnp.float32),
            jax.ShapeDtypeStruct((G, D), jnp.float32),
        ],
    )(parts, wt, a, batch3)


def kernel(feat, edge_index, batch_indices, edge_weight, W0, W1, a0, a1):
    row = edge_index[0]
    col = edge_index[1]
    batch3 = batch_indices.reshape(NB, 1, BN)
    a0r = a0.reshape(1, 1)
    a1r = a1.reshape(1, 1)

    parts0 = _sc_aggregate(feat, row, col, edge_weight)
    h0, hg0 = _tc_transform(parts0, W0.T, a0r, batch3)
    parts1 = _sc_aggregate(h0, row, col, edge_weight)
    h1, hg1 = _tc_transform(parts1, W1.T, a1r, batch3)
    hg = jnp.concatenate((hg0, hg1), axis=-1)
    return (h1, hg)


# speedup vs baseline: 3.6137x; 3.6137x over previous
import dataclasses
import functools

import jax
import jax.numpy as jnp
from jax import lax
from jax.experimental import pallas as pl
from jax.experimental.pallas import tpu as pltpu
from jax.experimental.pallas import tpu_sc as plsc

N = 10000
E = 320000
D = 128
G = 64

NC = 2
NS = 16
NW = NC * NS
EPW = E // NW
K = 80
NCHUNK = EPW // K
WB_STRIDE = 624
WB_SIZE = 640
ZR = 128

BN = 1000
NB = N // BN


def _sc_aggregate(x, row, col, w):
    mesh = plsc.VectorSubcoreMesh(core_axis_name="c", subcore_axis_name="s")
    cp = pltpu.CompilerParams()
    if "needs_layout_passes" in pltpu.CompilerParams.__dataclass_fields__:
        cp = dataclasses.replace(cp, needs_layout_passes=False)

    @functools.partial(
        pl.kernel,
        compiler_params=cp,
        out_type=jax.ShapeDtypeStruct((NC, N, D), jnp.float32),
        mesh=mesh,
        scratch_types=[
            pltpu.VMEM((K,), jnp.int32),
            pltpu.VMEM((K,), jnp.int32),
            pltpu.VMEM((K,), jnp.float32),
            pltpu.VMEM((K, D), jnp.float32),
            pltpu.VMEM((ZR, D), jnp.float32),
            pltpu.VMEM_SHARED((N, D), jnp.float32),
            pltpu.SemaphoreType.DMA,
        ],
    )
    def agg_kernel(x_hbm, row_hbm, col_hbm, w_hbm, out_hbm,
                   row_v, col_v, w_v, rows_v, zero_v, acc_sh, sem):
        cid = lax.axis_index("c")
        sid = lax.axis_index("s")
        wid = sid * NC + cid

        @pl.loop(0, ZR)
        def _(r):
            for dg in range(D // 16):
                zero_v[r, pl.ds(dg * 16, 16)] = jnp.zeros((16,), jnp.float32)

        wb_base = pl.multiple_of(sid * WB_STRIDE, 8)
        for j in range(WB_SIZE // ZR):
            pltpu.sync_copy(zero_v, acc_sh.at[pl.ds(wb_base + j * ZR, ZR)])
        plsc.subcore_barrier()

        base = wid * EPW

        @pl.loop(0, NCHUNK)
        def _(g):
            off = base + g * K
            pltpu.sync_copy(row_hbm.at[pl.ds(off, K)], row_v)
            pltpu.sync_copy(col_hbm.at[pl.ds(off, K)], col_v)
            pltpu.sync_copy(w_hbm.at[pl.ds(off, K)], w_v)
            pltpu.async_copy(x_hbm.at[row_v], rows_v, sem).wait()

            @pl.loop(0, K)
            def _(e):
                wvec = plsc.load_gather(w_v, [jnp.full((16,), e, jnp.int32)])
                for dg in range(D // 16):
                    sl = (e, pl.ds(dg * 16, 16))
                    rows_v[sl] = rows_v[sl] * wvec

            pltpu.sync_copy(rows_v, acc_sh.at[col_v], add=True)

        plsc.subcore_barrier()
        pltpu.sync_copy(acc_sh.at[pl.ds(wb_base, WB_SIZE)],
                        out_hbm.at[cid].at[pl.ds(wb_base, WB_SIZE)])

    return agg_kernel(x, row, col, w)


def _tc_transform_body(p_ref, wt_ref, a_ref, b_ref, h_ref, hg_ref):
    i = pl.program_id(0)
    x = p_ref[0] + p_ref[1]
    y = jnp.dot(x, wt_ref[...], preferred_element_type=jnp.float32)
    a = a_ref[0, 0]
    h = jnp.where(y >= 0, y, a * y)
    h_ref[...] = h
    labels = b_ref[0]
    onehot = (lax.broadcasted_iota(jnp.int32, (G, BN), 0) == labels
              ).astype(jnp.float32)
    contrib = jnp.dot(onehot, h, preferred_element_type=jnp.float32)

    @pl.when(i == 0)
    def _():
        hg_ref[...] = jnp.zeros_like(hg_ref)

    hg_ref[...] += contrib


def _tc_transform(parts, wt, a, batch3):
    return pl.pallas_call(
        _tc_transform_body,
        grid=(NB,),
        in_specs=[
            pl.BlockSpec((NC, BN, D), lambda i: (0, i, 0)),
            pl.BlockSpec((D, D), lambda i: (0, 0)),
            pl.BlockSpec((1, 1), lambda i: (0, 0)),
            pl.BlockSpec((1, 1, BN), lambda i: (i, 0, 0)),
        ],
        out_specs=[
            pl.BlockSpec((BN, D), lambda i: (i, 0)),
            pl.BlockSpec((G, D), lambda i: (0, 0)),
        ],
        out_shape=[
            jax.ShapeDtypeStruct((N, D), jnp.float32),
            jax.ShapeDtypeStruct((G, D), jnp.float32),
        ],
    )(parts, wt, a, batch3)


def kernel(feat, edge_index, batch_indices, edge_weight, W0, W1, a0, a1):
    row = edge_index[0]
    col = edge_index[1]
    batch3 = batch_indices.reshape(NB, 1, BN)
    a0r = a0.reshape(1, 1)
    a1r = a1.reshape(1, 1)

    parts0 = _sc_aggregate(feat, row, col, edge_weight)
    h0, hg0 = _tc_transform(parts0, W0.T, a0r, batch3)
    parts1 = _sc_aggregate(h0, row, col, edge_weight)
    h1, hg1 = _tc_transform(parts1, W1.T, a1r, batch3)
    hg = jnp.concatenate((hg0, hg1), axis=-1)
    return (h1, hg)


# speedup vs baseline: 4.0686x; 1.1259x over previous
import dataclasses
import functools

import jax
import jax.numpy as jnp
from jax import lax
from jax.experimental import pallas as pl
from jax.experimental.pallas import tpu as pltpu
from jax.experimental.pallas import tpu_sc as plsc

N = 10000
E = 320000
D = 128
G = 64

NC = 2
NS = 16
NW = NC * NS
K = 80
NCHUNK = 128
EPW = NCHUNK * K
E_PAD = NW * EPW
PRE = 8
WB_STRIDE = 624
WB_SIZE = 640

BN = 1000
NB = N // BN


def _sc_aggregate(x, row_p, col_p, w_p):
    mesh = plsc.VectorSubcoreMesh(core_axis_name="c", subcore_axis_name="s")
    cp = pltpu.CompilerParams()
    if "needs_layout_passes" in pltpu.CompilerParams.__dataclass_fields__:
        cp = dataclasses.replace(cp, needs_layout_passes=False)

    @functools.partial(
        pl.kernel,
        compiler_params=cp,
        out_type=jax.ShapeDtypeStruct((NC, N, D), jnp.float32),
        mesh=mesh,
        scratch_types=[
            [pltpu.VMEM((K,), jnp.int32) for _ in range(4)],
            [pltpu.VMEM((K,), jnp.int32) for _ in range(4)],
            [pltpu.VMEM((K,), jnp.float32) for _ in range(4)],
            [pltpu.VMEM((K, D), jnp.float32) for _ in range(2)],
            [pltpu.VMEM((K, D), jnp.float32) for _ in range(2)],
            pltpu.VMEM_SHARED((N, D), jnp.float32),
            [pltpu.SemaphoreType.DMA for _ in range(4)],
            [pltpu.SemaphoreType.DMA for _ in range(2)],
            [pltpu.SemaphoreType.DMA for _ in range(2)],
        ],
    )
    def agg_kernel(x_hbm, row_hbm, col_hbm, w_hbm, out_hbm,
                   rowb, colb, wb, rows, msg, acc_sh, isem, gsem, ssem):
        cid = lax.axis_index("c")
        sid = lax.axis_index("s")
        wid = sid * NC + cid
        base = pl.multiple_of(wid * EPW, 8)

        @pl.loop(0, K)
        def _(r):
            for dg in range(D // 16):
                msg[0][r, pl.ds(dg * 16, 16)] = jnp.zeros((16,), jnp.float32)

        wb_base = pl.multiple_of(sid * WB_STRIDE, 8)
        for j in range(WB_SIZE // K):
            pltpu.sync_copy(msg[0], acc_sh.at[pl.ds(wb_base + j * K, K)])

        def idx_descs(g, q):
            off = pl.multiple_of(base + g * K, 8)
            return (
                pltpu.make_async_copy(row_hbm.at[pl.ds(off, K)], rowb[q],
                                      isem[q]),
                pltpu.make_async_copy(col_hbm.at[pl.ds(off, K)], colb[q],
                                      isem[q]),
                pltpu.make_async_copy(w_hbm.at[pl.ds(off, K)], wb[q],
                                      isem[q]),
            )

        def gather_desc(q, pb):
            return pltpu.make_async_copy(
                x_hbm.at[rowb[q]], rows[pb], gsem[pb])

        def scatter_desc(q, pb):
            return pltpu.make_async_copy(
                msg[pb], acc_sh.at[colb[q]], ssem[pb])

        for g0 in (0, 1):
            for d in idx_descs(g0, g0):
                d.start()
        for g0 in (0, 1):
            for d in idx_descs(g0, g0):
                d.wait()
            gather_desc(g0, g0).start()

        plsc.subcore_barrier()

        @pl.loop(0, NCHUNK // 4)
        def _(t):
            for p in range(4):
                g = 4 * t + p
                pb = p % 2
                q2 = (p + 2) % 4

                gather_desc(p, pb).wait()

                @pl.when(g >= 2)
                def _():
                    scatter_desc(q2, pb).wait()

                @pl.when(g + 2 < NCHUNK)
                def _():
                    for d in idx_descs(g + 2, q2):
                        d.start()

                @pl.loop(0, K)
                def _(e):
                    wvec = plsc.load_gather(
                        wb[p], [jnp.full((16,), e, jnp.int32)])
                    for dg in range(D // 16):
                        sl = (e, pl.ds(dg * 16, 16))
                        msg[pb][sl] = rows[pb][sl] * wvec

                @pl.when(g + 2 < NCHUNK)
                def _():
                    for d in idx_descs(g + 2, q2):
                        d.wait()
                    gather_desc(q2, pb).start()

                pltpu.async_copy(msg[pb], acc_sh.at[colb[p]],
                                 ssem[pb], add=True)

        scatter_desc(2, 0).wait()
        scatter_desc(3, 1).wait()

        plsc.subcore_barrier()
        pltpu.sync_copy(acc_sh.at[pl.ds(wb_base, WB_SIZE)],
                        out_hbm.at[cid].at[pl.ds(wb_base, WB_SIZE)])

    return agg_kernel(x, row_p, col_p, w_p)


def _tc_transform_body(p_ref, wt_ref, a_ref, b_ref, h_ref, hg_ref):
    i = pl.program_id(0)
    x = p_ref[0] + p_ref[1]
    y = jnp.dot(x, wt_ref[...], preferred_element_type=jnp.float32)
    a = a_ref[0, 0]
    h = jnp.where(y >= 0, y, a * y)
    h_ref[...] = h
    labels = b_ref[0]
    onehot = (lax.broadcasted_iota(jnp.int32, (G, BN), 0) == labels
              ).astype(jnp.float32)
    contrib = jnp.dot(onehot, h, preferred_element_type=jnp.float32)

    @pl.when(i == 0)
    def _():
        hg_ref[...] = jnp.zeros_like(hg_ref)

    hg_ref[...] += contrib


def _tc_transform(parts, wt, a, batch3):
    return pl.pallas_call(
        _tc_transform_body,
        grid=(NB,),
        in_specs=[
            pl.BlockSpec((NC, BN, D), lambda i: (0, i, 0)),
            pl.BlockSpec((D, D), lambda i: (0, 0)),
            pl.BlockSpec((1, 1), lambda i: (0, 0)),
            pl.BlockSpec((1, 1, BN), lambda i: (i, 0, 0)),
        ],
        out_specs=[
            pl.BlockSpec((BN, D), lambda i: (i, 0)),
            pl.BlockSpec((G, D), lambda i: (0, 0)),
        ],
        out_shape=[
            jax.ShapeDtypeStruct((N, D), jnp.float32),
            jax.ShapeDtypeStruct((G, D), jnp.float32),
        ],
    )(parts, wt, a, batch3)


def kernel(feat, edge_index, batch_indices, edge_weight, W0, W1, a0, a1):
    npad = E_PAD - E
    pad_idx = (jnp.arange(npad, dtype=jnp.int32) * 37) % N
    row_p = jnp.concatenate((edge_index[0], pad_idx))
    col_p = jnp.concatenate((edge_index[1], pad_idx))
    w_p = jnp.concatenate((edge_weight, jnp.zeros((npad,), jnp.float32)))

    batch3 = batch_indices.reshape(NB, 1, BN)
    a0r = a0.reshape(1, 1)
    a1r = a1.reshape(1, 1)

    parts0 = _sc_aggregate(feat, row_p, col_p, w_p)
    h0, hg0 = _tc_transform(parts0, W0.T, a0r, batch3)
    parts1 = _sc_aggregate(h0, row_p, col_p, w_p)
    h1, hg1 = _tc_transform(parts1, W1.T, a1r, batch3)
    hg = jnp.concatenate((hg0, hg1), axis=-1)
    return (h1, hg)


# speedup vs baseline: 8.7810x; 2.1583x over previous
import dataclasses
import functools

import jax
import jax.numpy as jnp
from jax import lax
from jax.experimental import pallas as pl
from jax.experimental.pallas import tpu as pltpu
from jax.experimental.pallas import tpu_sc as plsc

N = 10000
E = 320000
D = 128
G = 64

NC = 2
NS = 16
NW = NC * NS
K = 80
NCHUNK = 128
EPW = NCHUNK * K
E_PAD = NW * EPW
PRE = 8
WB_STRIDE = 624
WB_SIZE = 640

BN = 1000
NB = N // BN

_GATHER_DN = lax.GatherDimensionNumbers(
    offset_dims=(), collapsed_slice_dims=(0,), start_index_map=(0,))


def _lane_bcast(v16, j):
    return lax.gather(v16, jnp.full((16, 1), j, jnp.int32), _GATHER_DN,
                      slice_sizes=(1,),
                      mode=lax.GatherScatterMode.PROMISE_IN_BOUNDS)


def _sc_aggregate(x, row_p, col_p, w_p):
    mesh = plsc.VectorSubcoreMesh(core_axis_name="c", subcore_axis_name="s")
    cp = pltpu.CompilerParams()
    if "needs_layout_passes" in pltpu.CompilerParams.__dataclass_fields__:
        cp = dataclasses.replace(cp, needs_layout_passes=False)

    @functools.partial(
        pl.kernel,
        compiler_params=cp,
        out_type=jax.ShapeDtypeStruct((NC, N, D), jnp.float32),
        mesh=mesh,
        scratch_types=[
            [pltpu.VMEM((K,), jnp.int32) for _ in range(4)],
            [pltpu.VMEM((K,), jnp.int32) for _ in range(4)],
            [pltpu.VMEM((K,), jnp.float32) for _ in range(4)],
            [pltpu.VMEM((K, D), jnp.float32) for _ in range(2)],
            [pltpu.VMEM((K, D), jnp.float32) for _ in range(2)],
            pltpu.VMEM_SHARED((N, D), jnp.float32),
            [pltpu.SemaphoreType.DMA for _ in range(4)],
            [pltpu.SemaphoreType.DMA for _ in range(2)],
            [pltpu.SemaphoreType.DMA for _ in range(2)],
        ],
    )
    def agg_kernel(x_hbm, row_hbm, col_hbm, w_hbm, out_hbm,
                   rowb, colb, wb, rows, msg, acc_sh, isem, gsem, ssem):
        cid = lax.axis_index("c")
        sid = lax.axis_index("s")
        wid = sid * NC + cid
        base = pl.multiple_of(wid * EPW, 8)

        @pl.loop(0, K)
        def _(r):
            for dg in range(D // 16):
                msg[0][r, pl.ds(dg * 16, 16)] = jnp.zeros((16,), jnp.float32)

        wb_base = pl.multiple_of(sid * WB_STRIDE, 8)
        for j in range(WB_SIZE // K):
            pltpu.sync_copy(msg[0], acc_sh.at[pl.ds(wb_base + j * K, K)])

        def idx_descs(g, q):
            off = pl.multiple_of(base + g * K, 8)
            return (
                pltpu.make_async_copy(row_hbm.at[pl.ds(off, K)], rowb[q],
                                      isem[q]),
                pltpu.make_async_copy(col_hbm.at[pl.ds(off, K)], colb[q],
                                      isem[q]),
                pltpu.make_async_copy(w_hbm.at[pl.ds(off, K)], wb[q],
                                      isem[q]),
            )

        def gather_desc(q, pb):
            return pltpu.make_async_copy(
                x_hbm.at[rowb[q]], rows[pb], gsem[pb])

        def scatter_desc(q, pb):
            return pltpu.make_async_copy(
                msg[pb], acc_sh.at[colb[q]], ssem[pb])

        for g0 in (0, 1):
            for d in idx_descs(g0, g0):
                d.start()
        for g0 in (0, 1):
            for d in idx_descs(g0, g0):
                d.wait()
            gather_desc(g0, g0).start()

        plsc.subcore_barrier()

        @pl.loop(0, NCHUNK // 4)
        def _(t):
            for p in range(4):
                g = 4 * t + p
                pb = p % 2
                q2 = (p + 2) % 4

                gather_desc(p, pb).wait()

                @pl.when(g >= 2)
                def _():
                    scatter_desc(q2, pb).wait()

                @pl.when(g + 2 < NCHUNK)
                def _():
                    for d in idx_descs(g + 2, q2):
                        d.start()

                for b in range(K // 16):
                    w16 = wb[p][pl.ds(16 * b, 16)]
                    for j in range(16):
                        wvec = _lane_bcast(w16, j)
                        e = 16 * b + j
                        for dg in range(D // 16):
                            sl = (e, pl.ds(dg * 16, 16))
                            msg[pb][sl] = rows[pb][sl] * wvec

                @pl.when(g + 2 < NCHUNK)
                def _():
                    for d in idx_descs(g + 2, q2):
                        d.wait()
                    gather_desc(q2, pb).start()

                pltpu.async_copy(msg[pb], acc_sh.at[colb[p]],
                                 ssem[pb], add=True)

        scatter_desc(2, 0).wait()
        scatter_desc(3, 1).wait()

        plsc.subcore_barrier()
        pltpu.sync_copy(acc_sh.at[pl.ds(wb_base, WB_SIZE)],
                        out_hbm.at[cid].at[pl.ds(wb_base, WB_SIZE)])

    return agg_kernel(x, row_p, col_p, w_p)


def _tc_transform_body(p_ref, wt_ref, a_ref, b_ref, h_ref, hg_ref):
    i = pl.program_id(0)
    x = p_ref[0] + p_ref[1]
    y = jnp.dot(x, wt_ref[...], preferred_element_type=jnp.float32)
    a = a_ref[0, 0]
    h = jnp.where(y >= 0, y, a * y)
    h_ref[...] = h
    labels = b_ref[0]
    onehot = (lax.broadcasted_iota(jnp.int32, (G, BN), 0) == labels
              ).astype(jnp.float32)
    contrib = jnp.dot(onehot, h, preferred_element_type=jnp.float32)

    @pl.when(i == 0)
    def _():
        hg_ref[...] = jnp.zeros_like(hg_ref)

    hg_ref[...] += contrib


def _tc_transform(parts, wt, a, batch3):
    return pl.pallas_call(
        _tc_transform_body,
        grid=(NB,),
        in_specs=[
            pl.BlockSpec((NC, BN, D), lambda i: (0, i, 0)),
            pl.BlockSpec((D, D), lambda i: (0, 0)),
            pl.BlockSpec((1, 1), lambda i: (0, 0)),
            pl.BlockSpec((1, 1, BN), lambda i: (i, 0, 0)),
        ],
        out_specs=[
            pl.BlockSpec((BN, D), lambda i: (i, 0)),
            pl.BlockSpec((G, D), lambda i: (0, 0)),
        ],
        out_shape=[
            jax.ShapeDtypeStruct((N, D), jnp.float32),
            jax.ShapeDtypeStruct((G, D), jnp.float32),
        ],
    )(parts, wt, a, batch3)


def kernel(feat, edge_index, batch_indices, edge_weight, W0, W1, a0, a1):
    npad = E_PAD - E
    pad_idx = (jnp.arange(npad, dtype=jnp.int32) * 37) % N
    row_p = jnp.concatenate((edge_index[0], pad_idx))
    col_p = jnp.concatenate((edge_index[1], pad_idx))
    w_p = jnp.concatenate((edge_weight, jnp.zeros((npad,), jnp.float32)))

    batch3 = batch_indices.reshape(NB, 1, BN)
    a0r = a0.reshape(1, 1)
    a1r = a1.reshape(1, 1)

    parts0 = _sc_aggregate(feat, row_p, col_p, w_p)
    h0, hg0 = _tc_transform(parts0, W0.T, a0r, batch3)
    parts1 = _sc_aggregate(h0, row_p, col_p, w_p)
    h1, hg1 = _tc_transform(parts1, W1.T, a1r, batch3)
    hg = jnp.concatenate((hg0, hg1), axis=-1)
    return (h1, hg)


# speedup vs baseline: 9.6447x; 1.0984x over previous
import dataclasses
import functools

import jax
import jax.numpy as jnp
from jax import lax
from jax.experimental import pallas as pl
from jax.experimental.pallas import tpu as pltpu
from jax.experimental.pallas import tpu_sc as plsc

N = 10000
E = 320000
D = 128
G = 64

NC = 2
NS = 16
NW = NC * NS
K = 64
NCHUNK = 160
EPW = NCHUNK * K
E_PAD = NW * EPW
PRE = 8
WB_STRIDE = 624
WB_SIZE = 640

BN = 1000
NB = N // BN

_GATHER_DN = lax.GatherDimensionNumbers(
    offset_dims=(), collapsed_slice_dims=(0,), start_index_map=(0,))


def _pack_bf16(x):
    xb = x.astype(jnp.bfloat16).reshape(-1, D // 32, 2, 16)
    pairs = jnp.stack((xb[:, :, 0, :], xb[:, :, 1, :]), axis=-1)
    packed = jax.lax.bitcast_convert_type(pairs, jnp.int32)
    return packed.reshape(-1, D // 2)


def _lane_bcast(v16, j):
    return lax.gather(v16, jnp.full((16, 1), j, jnp.int32), _GATHER_DN,
                      slice_sizes=(1,),
                      mode=lax.GatherScatterMode.PROMISE_IN_BOUNDS)


def _sc_aggregate(x, row_p, col_p, w_p):
    mesh = plsc.VectorSubcoreMesh(core_axis_name="c", subcore_axis_name="s")
    cp = pltpu.CompilerParams()
    if "needs_layout_passes" in pltpu.CompilerParams.__dataclass_fields__:
        cp = dataclasses.replace(cp, needs_layout_passes=False)

    @functools.partial(
        pl.kernel,
        compiler_params=cp,
        out_type=jax.ShapeDtypeStruct((NC, N, D), jnp.float32),
        mesh=mesh,
        scratch_types=[
            [pltpu.VMEM((K,), jnp.int32) for _ in range(4)],
            [pltpu.VMEM((K,), jnp.int32) for _ in range(4)],
            [pltpu.VMEM((K,), jnp.int32) for _ in range(2)],
            [pltpu.VMEM((K,), jnp.float32) for _ in range(4)],
            [pltpu.VMEM((K, D), jnp.float32) for _ in range(4)],
            [pltpu.VMEM((K, D), jnp.float32) for _ in range(2)],
            pltpu.VMEM_SHARED((N, D), jnp.float32),
            [pltpu.SemaphoreType.DMA for _ in range(4)],
            [pltpu.SemaphoreType.DMA for _ in range(2)],
            [pltpu.SemaphoreType.DMA for _ in range(2)],
        ],
    )
    def agg_kernel(x_hbm, row_hbm, col_hbm, w_hbm, out_hbm,
                   rowb, colb, colS, wb, rows, msg, acc_sh,
                   isem, gsem, ssem):
        cid = lax.axis_index("c")
        sid = lax.axis_index("s")
        wid = sid * NC + cid
        base = pl.multiple_of(wid * EPW, 8)

        @pl.loop(0, K)
        def _(r):
            for dg in range(D // 16):
                msg[0][r, pl.ds(dg * 16, 16)] = jnp.zeros((16,), jnp.float32)

        wb_base = pl.multiple_of(sid * WB_STRIDE, 8)
        for j in range(WB_SIZE // K):
            pltpu.sync_copy(msg[0], acc_sh.at[pl.ds(wb_base + j * K, K)])

        def idx_descs(g, q):
            off = pl.multiple_of(base + g * K, 8)
            return (
                pltpu.make_async_copy(row_hbm.at[pl.ds(off, K)], rowb[q],
                                      isem[q]),
                pltpu.make_async_copy(col_hbm.at[pl.ds(off, K)], colb[q],
                                      isem[q]),
                pltpu.make_async_copy(w_hbm.at[pl.ds(off, K)], wb[q],
                                      isem[q]),
            )

        def gather_desc(q, pb):
            return pltpu.make_async_copy(
                x_hbm.at[rowb[q]], rows[q], gsem[pb])

        def gather_start(q, pb):
            gather_desc(q, pb).start()

        def gather_wait(q, pb):
            gather_desc(q, pb).wait()

        def scatter_desc(pb):
            return pltpu.make_async_copy(
                msg[pb], acc_sh.at[colS[pb]], ssem[pb])

        for g0 in (0, 1, 2):
            for d in idx_descs(g0, g0):
                d.start()
        for g0 in (0, 1):
            for d in idx_descs(g0, g0):
                d.wait()
            gather_start(g0, g0)

        plsc.subcore_barrier()

        @pl.loop(0, NCHUNK // 4)
        def _(t):
            for p in range(4):
                g = 4 * t + p
                pb = p % 2
                q2 = (p + 2) % 4
                q3 = (p + 3) % 4

                gather_wait(p, pb)

                @pl.when(g >= 2)
                def _():
                    scatter_desc(pb).wait()

                @pl.when(g + 2 < NCHUNK)
                def _():
                    for d in idx_descs(g + 2, q2):
                        d.wait()
                    gather_start(q2, pb)

                @pl.when(g + 3 < NCHUNK)
                def _():
                    for d in idx_descs(g + 3, q3):
                        d.start()

                for i in range(K // 16):
                    colS[pb][pl.ds(16 * i, 16)] = colb[p][pl.ds(16 * i, 16)]

                for b in range(K // 16):
                    w16 = wb[p][pl.ds(16 * b, 16)]
                    for j in range(16):
                        wvec = _lane_bcast(w16, j)
                        e = 16 * b + j
                        for dg in range(D // 16):
                            sl = (e, pl.ds(dg * 16, 16))
                            msg[pb][sl] = rows[p][sl] * wvec

                pltpu.async_copy(msg[pb], acc_sh.at[colS[pb]],
                                 ssem[pb], add=True)

        scatter_desc(0).wait()
        scatter_desc(1).wait()

        plsc.subcore_barrier()
        pltpu.sync_copy(acc_sh.at[pl.ds(wb_base, WB_SIZE)],
                        out_hbm.at[cid].at[pl.ds(wb_base, WB_SIZE)])

    return agg_kernel(x, row_p, col_p, w_p)


def _tc_transform_body(p_ref, wt_ref, a_ref, b_ref, h_ref, hg_ref):
    i = pl.program_id(0)
    x = p_ref[0] + p_ref[1]
    y = jnp.dot(x, wt_ref[...], preferred_element_type=jnp.float32)
    a = a_ref[0, 0]
    h = jnp.where(y >= 0, y, a * y)
    h_ref[...] = h
    labels = b_ref[0]
    onehot = (lax.broadcasted_iota(jnp.int32, (G, BN), 0) == labels
              ).astype(jnp.float32)
    contrib = jnp.dot(onehot, h, preferred_element_type=jnp.float32)

    @pl.when(i == 0)
    def _():
        hg_ref[...] = jnp.zeros_like(hg_ref)

    hg_ref[...] += contrib


def _tc_transform(parts, wt, a, batch3):
    return pl.pallas_call(
        _tc_transform_body,
        grid=(NB,),
        in_specs=[
            pl.BlockSpec((NC, BN, D), lambda i: (0, i, 0)),
            pl.BlockSpec((D, D), lambda i: (0, 0)),
            pl.BlockSpec((1, 1), lambda i: (0, 0)),
            pl.BlockSpec((1, 1, BN), lambda i: (i, 0, 0)),
        ],
        out_specs=[
            pl.BlockSpec((BN, D), lambda i: (i, 0)),
            pl.BlockSpec((G, D), lambda i: (0, 0)),
        ],
        out_shape=[
            jax.ShapeDtypeStruct((N, D), jnp.float32),
            jax.ShapeDtypeStruct((G, D), jnp.float32),
        ],
    )(parts, wt, a, batch3)


def kernel(feat, edge_index, batch_indices, edge_weight, W0, W1, a0, a1):
    npad = E_PAD - E
    pad_idx = (jnp.arange(npad, dtype=jnp.int32) * 37) % N
    row_p = jnp.concatenate((edge_index[0], pad_idx))
    col_p = jnp.concatenate((edge_index[1], pad_idx))
    w_p = jnp.concatenate((edge_weight, jnp.zeros((npad,), jnp.float32)))

    batch3 = batch_indices.reshape(NB, 1, BN)
    a0r = a0.reshape(1, 1)
    a1r = a1.reshape(1, 1)

    parts0 = _sc_aggregate(feat, row_p, col_p, w_p)
    h0, hg0 = _tc_transform(parts0, W0.T, a0r, batch3)
    parts1 = _sc_aggregate(h0, row_p, col_p, w_p)
    h1, hg1 = _tc_transform(parts1, W1.T, a1r, batch3)
    hg = jnp.concatenate((hg0, hg1), axis=-1)
    return (h1, hg)


# speedup vs baseline: 9.7475x; 1.0107x over previous
import dataclasses
import functools

import jax
import jax.numpy as jnp
from jax import lax
from jax.experimental import pallas as pl
from jax.experimental.pallas import tpu as pltpu
from jax.experimental.pallas import tpu_sc as plsc

N = 10000
E = 320000
D = 128
G = 64

NC = 2
NS = 16
NW = NC * NS
K = 64
NCHUNK = 160
EPW = NCHUNK * K
E_PAD = NW * EPW
PRE = 8
WB_STRIDE = 624
WB_SIZE = 640

BN = 2000
NB = N // BN

_GATHER_DN = lax.GatherDimensionNumbers(
    offset_dims=(), collapsed_slice_dims=(0,), start_index_map=(0,))


def _pack_bf16(x):
    xb = x.astype(jnp.bfloat16).reshape(-1, D // 32, 2, 16)
    pairs = jnp.stack((xb[:, :, 0, :], xb[:, :, 1, :]), axis=-1)
    packed = jax.lax.bitcast_convert_type(pairs, jnp.int32)
    return packed.reshape(-1, D // 2)


def _lane_bcast(v16, j):
    return lax.gather(v16, jnp.full((16, 1), j, jnp.int32), _GATHER_DN,
                      slice_sizes=(1,),
                      mode=lax.GatherScatterMode.PROMISE_IN_BOUNDS)


def _sc_aggregate(x, row_p, col_p, w_p):
    mesh = plsc.VectorSubcoreMesh(core_axis_name="c", subcore_axis_name="s")
    cp = pltpu.CompilerParams()
    if "needs_layout_passes" in pltpu.CompilerParams.__dataclass_fields__:
        cp = dataclasses.replace(cp, needs_layout_passes=False)

    @functools.partial(
        pl.kernel,
        compiler_params=cp,
        out_type=jax.ShapeDtypeStruct((NC, N, D), jnp.float32),
        mesh=mesh,
        scratch_types=[
            [pltpu.VMEM((K,), jnp.int32) for _ in range(4)],
            [pltpu.VMEM((K,), jnp.int32) for _ in range(4)],
            [pltpu.VMEM((K,), jnp.int32) for _ in range(2)],
            [pltpu.VMEM((K,), jnp.float32) for _ in range(4)],
            [pltpu.VMEM((K, D), jnp.float32) for _ in range(4)],
            [pltpu.VMEM((K, D), jnp.float32) for _ in range(2)],
            pltpu.VMEM_SHARED((N, D), jnp.float32),
            [pltpu.SemaphoreType.DMA for _ in range(4)],
            [pltpu.SemaphoreType.DMA for _ in range(2)],
            [pltpu.SemaphoreType.DMA for _ in range(2)],
        ],
    )
    def agg_kernel(x_hbm, row_hbm, col_hbm, w_hbm, out_hbm,
                   rowb, colb, colS, wb, rows, msg, acc_sh,
                   isem, gsem, ssem):
        cid = lax.axis_index("c")
        sid = lax.axis_index("s")
        wid = sid * NC + cid
        base = pl.multiple_of(wid * EPW, 8)

        @pl.loop(0, K)
        def _(r):
            for dg in range(D // 16):
                msg[0][r, pl.ds(dg * 16, 16)] = jnp.zeros((16,), jnp.float32)

        wb_base = pl.multiple_of(sid * WB_STRIDE, 8)
        for j in range(WB_SIZE // K):
            pltpu.sync_copy(msg[0], acc_sh.at[pl.ds(wb_base + j * K, K)])

        def idx_descs(g, q):
            off = pl.multiple_of(base + g * K, 8)
            return (
                pltpu.make_async_copy(row_hbm.at[pl.ds(off, K)], rowb[q],
                                      isem[q]),
                pltpu.make_async_copy(col_hbm.at[pl.ds(off, K)], colb[q],
                                      isem[q]),
                pltpu.make_async_copy(w_hbm.at[pl.ds(off, K)], wb[q],
                                      isem[q]),
            )

        def gather_desc(q, pb):
            return pltpu.make_async_copy(
                x_hbm.at[rowb[q]], rows[q], gsem[pb])

        def gather_start(q, pb):
            gather_desc(q, pb).start()

        def gather_wait(q, pb):
            gather_desc(q, pb).wait()

        def scatter_desc(pb):
            return pltpu.make_async_copy(
                msg[pb], acc_sh.at[colS[pb]], ssem[pb])

        for g0 in (0, 1, 2):
            for d in idx_descs(g0, g0):
                d.start()
        for g0 in (0, 1):
            for d in idx_descs(g0, g0):
                d.wait()
            gather_start(g0, g0)

        plsc.subcore_barrier()

        @pl.loop(0, NCHUNK // 4)
        def _(t):
            for p in range(4):
                g = 4 * t + p
                pb = p % 2
                q2 = (p + 2) % 4
                q3 = (p + 3) % 4

                gather_wait(p, pb)

                @pl.when(g + 2 < NCHUNK)
                def _():
                    for d in idx_descs(g + 2, q2):
                        d.wait()
                    gather_start(q2, pb)

                @pl.when(g + 3 < NCHUNK)
                def _():
                    for d in idx_descs(g + 3, q3):
                        d.start()

                @pl.when(g >= 2)
                def _():
                    scatter_desc(pb).wait()

                for i in range(K // 16):
                    colS[pb][pl.ds(16 * i, 16)] = colb[p][pl.ds(16 * i, 16)]

                for b in range(K // 16):
                    w16 = wb[p][pl.ds(16 * b, 16)]
                    for j in range(16):
                        wvec = _lane_bcast(w16, j)
                        e = 16 * b + j
                        for dg in range(D // 16):
                            sl = (e, pl.ds(dg * 16, 16))
                            msg[pb][sl] = rows[p][sl] * wvec

                pltpu.async_copy(msg[pb], acc_sh.at[colS[pb]],
                                 ssem[pb], add=True)

        scatter_desc(0).wait()
        scatter_desc(1).wait()

        plsc.subcore_barrier()
        pltpu.sync_copy(acc_sh.at[pl.ds(wb_base, WB_SIZE)],
                        out_hbm.at[cid].at[pl.ds(wb_base, WB_SIZE)])

    return agg_kernel(x, row_p, col_p, w_p)


def _tc_transform_body(p_ref, wt_ref, a_ref, b_ref, h_ref, hg_ref):
    i = pl.program_id(0)
    x = p_ref[0] + p_ref[1]
    y = jnp.dot(x, wt_ref[...], preferred_element_type=jnp.float32)
    a = a_ref[0, 0]
    h = jnp.where(y >= 0, y, a * y)
    h_ref[...] = h
    labels = b_ref[0]
    onehot = (lax.broadcasted_iota(jnp.int32, (G, BN), 0) == labels
              ).astype(jnp.float32)
    contrib = jnp.dot(onehot, h, preferred_element_type=jnp.float32)

    @pl.when(i == 0)
    def _():
        hg_ref[...] = jnp.zeros_like(hg_ref)

    hg_ref[...] += contrib


def _tc_transform(parts, wt, a, batch3):
    return pl.pallas_call(
        _tc_transform_body,
        grid=(NB,),
        in_specs=[
            pl.BlockSpec((NC, BN, D), lambda i: (0, i, 0)),
            pl.BlockSpec((D, D), lambda i: (0, 0)),
            pl.BlockSpec((1, 1), lambda i: (0, 0)),
            pl.BlockSpec((1, 1, BN), lambda i: (i, 0, 0)),
        ],
        out_specs=[
            pl.BlockSpec((BN, D), lambda i: (i, 0)),
            pl.BlockSpec((G, D), lambda i: (0, 0)),
        ],
        out_shape=[
            jax.ShapeDtypeStruct((N, D), jnp.float32),
            jax.ShapeDtypeStruct((G, D), jnp.float32),
        ],
    )(parts, wt, a, batch3)


def kernel(feat, edge_index, batch_indices, edge_weight, W0, W1, a0, a1):
    npad = E_PAD - E
    pad_idx = (jnp.arange(npad, dtype=jnp.int32) * 37) % N
    row_p = jnp.concatenate((edge_index[0], pad_idx))
    col_p = jnp.concatenate((edge_index[1], pad_idx))
    w_p = jnp.concatenate((edge_weight, jnp.zeros((npad,), jnp.float32)))

    batch3 = batch_indices.reshape(NB, 1, BN)
    a0r = a0.reshape(1, 1)
    a1r = a1.reshape(1, 1)

    parts0 = _sc_aggregate(feat, row_p, col_p, w_p)
    h0, hg0 = _tc_transform(parts0, W0.T, a0r, batch3)
    parts1 = _sc_aggregate(h0, row_p, col_p, w_p)
    h1, hg1 = _tc_transform(parts1, W1.T, a1r, batch3)
    hg = jnp.concatenate((hg0, hg1), axis=-1)
    return (h1, hg)
